# Initial kernel scaffold; baseline (speedup 1.0000x reference)
import jax
import jax.numpy as jnp
from jax.experimental import pallas as pl


def kernel(x, mask, source):
    raise NotImplementedError("write your pallas kernel here")


# speedup vs baseline: 1.3123x; 1.3123x over previous
import functools

import jax
import jax.numpy as jnp
from jax import lax
from jax.experimental import pallas as pl
from jax.experimental.pallas import tpu as pltpu
from jax.experimental.pallas import tpu_sc as plsc

L = 16
NC = 2
NS = 16
NW = NC * NS
MULT = 0x01010101


def _mesh():
    return plsc.VectorSubcoreMesh(core_axis_name="c", subcore_axis_name="s")


def _build_counts(n):
    wpc = n // 4 // NW

    @functools.partial(
        pl.kernel,
        mesh=_mesh(),
        out_type=jax.ShapeDtypeStruct((NW, L), jnp.int32),
        compiler_params=pltpu.CompilerParams(needs_layout_passes=False),
        scratch_types=[
            pltpu.VMEM((wpc,), jnp.int32),
            pltpu.VMEM((L,), jnp.int32),
        ],
    )
    def k(mw_hbm, cnt_hbm, mwv, cstage):
        wid = lax.axis_index("s") * NC + lax.axis_index("c")
        woff = pl.multiple_of(wid * wpc, 8)
        pltpu.sync_copy(mw_hbm.at[pl.ds(woff, wpc)], mwv)

        def count_body(i, acc):
            w = mwv[pl.ds(i * L, L)]
            return acc + lax.shift_right_logical(w * MULT, 24)

        acc = lax.fori_loop(0, wpc // L, count_body, jnp.zeros((L,), jnp.int32))
        cstage[...] = jnp.full((L,), jnp.sum(acc), jnp.int32)
        pltpu.sync_copy(cstage, cnt_hbm.at[wid])

    return k


def _build_apply(n):
    chunk = n // NW
    sub = 32768
    subw = sub // 4
    nsub = chunk // sub
    srcv_len = sub + 16

    @functools.partial(
        pl.kernel,
        mesh=_mesh(),
        out_type=jax.ShapeDtypeStruct((n,), jnp.float32),
        compiler_params=pltpu.CompilerParams(needs_layout_passes=False),
        scratch_types=[
            pltpu.VMEM((subw,), jnp.int32),
            pltpu.VMEM((srcv_len,), jnp.float32),
            pltpu.VMEM((sub,), jnp.float32),
            pltpu.VMEM((NW, L), jnp.int32),
        ],
    )
    def k(mw_hbm, x_hbm, src_hbm, cnt_hbm, out_hbm, mwv, srcv, outv, call_v):
        wid = lax.axis_index("s") * NC + lax.axis_index("c")
        pltpu.sync_copy(cnt_hbm, call_v)

        bacc = jnp.zeros((L,), jnp.int32)
        for j in range(NW):
            take = jnp.full((L,), (j < wid).astype(jnp.int32), jnp.int32)
            bacc = bacc + call_v[j] * take
        base = jnp.max(bacc)

        chunk_off = wid * chunk
        iota4 = lax.iota(jnp.int32, L) * 4

        def sub_body(b, pos):
            eoff = pl.multiple_of(chunk_off + b * sub, 8)
            pltpu.sync_copy(x_hbm.at[pl.ds(eoff, sub)], outv)
            woff = pl.multiple_of((chunk_off + b * sub) // 4, 8)
            pltpu.sync_copy(mw_hbm.at[pl.ds(woff, subw)], mwv)
            a = pl.multiple_of(jnp.minimum(pos & -8, n - srcv_len), 8)
            adj = pos - a
            pltpu.sync_copy(src_hbm.at[pl.ds(a, srcv_len)], srcv)

            def ibody(kk, c0):
                w = mwv[pl.ds(kk * L, L)]
                p = w * MULT
                t = lax.shift_right_logical(p, 24)
                excl = plsc.cumsum(t) - t
                eb = excl + c0
                i0 = p & 0xFF
                i1 = lax.shift_right_logical(p, 8) & 0xFF
                i2 = lax.shift_right_logical(p, 16) & 0xFF
                incs = (i0, i1, i2, t)
                ms = (i0 > 0, i1 > i0, i2 > i1, t > i2)
                xb = iota4 + kk * (4 * L)
                tot = None
                for j in range(4):
                    g = plsc.load_gather(srcv, [jnp.maximum(eb + incs[j], 0)])
                    plsc.store_scatter(outv, [xb + j], g, mask=ms[j])
                    pc = plsc.all_reduce_population_count(ms[j])
                    tot = pc if tot is None else tot + pc
                return c0 + tot

            c0f = lax.fori_loop(0, subw // L, ibody, jnp.full((L,), adj - 1, jnp.int32))
            pltpu.sync_copy(outv, out_hbm.at[pl.ds(eoff, sub)])
            return pos + (jnp.max(c0f) - (adj - 1))

        lax.fori_loop(0, nsub, sub_body, base)

    return k


def kernel(x, mask, source):
    n = x.size
    xf = x.reshape(-1)
    sf = source.reshape(-1)
    mw = lax.bitcast_convert_type(
        mask.reshape(-1, 4).astype(jnp.uint8), jnp.int32
    ).reshape(-1)
    cnt = _build_counts(n)(mw)
    out = _build_apply(n)(mw, xf, sf, cnt)
    return out.reshape(x.shape)


# speedup vs baseline: 1.3325x; 1.0154x over previous
import functools

import jax
import jax.numpy as jnp
from jax import lax
from jax.experimental import pallas as pl
from jax.experimental.pallas import tpu as pltpu
from jax.experimental.pallas import tpu_sc as plsc

L = 16
NC = 2
NS = 16
NW = NC * NS
MULT = 0x01010101


def _mesh():
    return plsc.VectorSubcoreMesh(core_axis_name="c", subcore_axis_name="s")


def _build_prefix(n):
    wpc = n // 4 // NW

    def body(mw_ref, pre_ref):
        w = mw_ref[...]
        t = (
            (w & 0xFF)
            + (lax.shift_right_logical(w, 8) & 0xFF)
            + (lax.shift_right_logical(w, 16) & 0xFF)
            + (lax.shift_right_logical(w, 24) & 0xFF)
        )
        red = jnp.sum(t, axis=1, keepdims=True)
        row = lax.broadcasted_iota(jnp.int32, (NW, NW), 0)
        col = lax.broadcasted_iota(jnp.int32, (NW, NW), 1)
        tri = (row > col).astype(jnp.float32)
        lo = (red & 0xFF).astype(jnp.float32)
        hi = lax.shift_right_logical(red, 8).astype(jnp.float32)
        dn = (((1,), (0,)), ((), ()))
        pre = (
            lax.dot_general(tri, lo, dn, preferred_element_type=jnp.float32)
            + 256.0 * lax.dot_general(tri, hi, dn, preferred_element_type=jnp.float32)
        )
        pre_ref[...] = jnp.broadcast_to(pre.astype(jnp.int32), (NW, 128))

    return pl.pallas_call(
        body,
        out_shape=jax.ShapeDtypeStruct((NW, 128), jnp.int32),
        grid=(),
    )


def _build_apply(n):
    chunk = n // NW
    sub = 32768
    subw = sub // 4
    nsub = chunk // sub
    srcv_len = sub + 16

    @functools.partial(
        pl.kernel,
        mesh=_mesh(),
        out_type=jax.ShapeDtypeStruct((n,), jnp.float32),
        compiler_params=pltpu.CompilerParams(needs_layout_passes=False),
        scratch_types=[
            pltpu.VMEM((subw,), jnp.int32),
            pltpu.VMEM((srcv_len,), jnp.float32),
            pltpu.VMEM((sub,), jnp.float32),
            pltpu.VMEM((L,), jnp.int32),
        ],
    )
    def k(mw_hbm, x_hbm, src_hbm, pre_hbm, out_hbm, mwv, srcv, outv, base_v):
        wid = lax.axis_index("s") * NC + lax.axis_index("c")
        poff = pl.multiple_of(wid * 128, 8)
        pltpu.sync_copy(pre_hbm.at[pl.ds(poff, L)], base_v)
        base = jnp.max(base_v[...])

        chunk_off = wid * chunk
        iota4 = lax.iota(jnp.int32, L) * 4

        def sub_body(b, pos):
            eoff = pl.multiple_of(chunk_off + b * sub, 8)
            pltpu.sync_copy(x_hbm.at[pl.ds(eoff, sub)], outv)
            woff = pl.multiple_of((chunk_off + b * sub) // 4, 8)
            pltpu.sync_copy(mw_hbm.at[pl.ds(woff, subw)], mwv)
            a = pl.multiple_of(jnp.minimum(pos & -8, n - srcv_len), 8)
            adj = pos - a
            pltpu.sync_copy(src_hbm.at[pl.ds(a, srcv_len)], srcv)

            def ibody(kk, c0):
                w = mwv[pl.ds(kk * L, L)]
                p = w * MULT
                t = lax.shift_right_logical(p, 24)
                excl = plsc.cumsum(t) - t
                eb = excl + c0
                i0 = p & 0xFF
                i1 = lax.shift_right_logical(p, 8) & 0xFF
                i2 = lax.shift_right_logical(p, 16) & 0xFF
                incs = (i0, i1, i2, t)
                ms = (i0 > 0, i1 > i0, i2 > i1, t > i2)
                xb = iota4 + kk * (4 * L)
                tot = None
                for j in range(4):
                    g = plsc.load_gather(srcv, [jnp.maximum(eb + incs[j], 0)])
                    plsc.store_scatter(outv, [xb + j], g, mask=ms[j])
                    pc = plsc.all_reduce_population_count(ms[j])
                    tot = pc if tot is None else tot + pc
                return c0 + tot

            c0f = lax.fori_loop(0, subw // L, ibody, jnp.full((L,), adj - 1, jnp.int32))
            pltpu.sync_copy(outv, out_hbm.at[pl.ds(eoff, sub)])
            return pos + (jnp.max(c0f) - (adj - 1))

        lax.fori_loop(0, nsub, sub_body, base)

    return k


def kernel(x, mask, source):
    n = x.size
    xf = x.reshape(-1)
    sf = source.reshape(-1)
    mw = lax.bitcast_convert_type(
        mask.reshape(-1, 4).astype(jnp.uint8), jnp.int32
    ).reshape(-1)
    pre = _build_prefix(n)(mw.reshape(NW, -1)).reshape(-1)
    out = _build_apply(n)(mw, xf, sf, pre)
    return out.reshape(x.shape)


# speedup vs baseline: 6.0830x; 4.5650x over previous
import functools

import jax
import jax.numpy as jnp
from jax import lax
from jax.experimental import pallas as pl
from jax.experimental.pallas import tpu as pltpu
from jax.experimental.pallas import tpu_sc as plsc

L = 16
NC = 2
NS = 16
NW = NC * NS
MULT = 0x01010101


def _mesh():
    return plsc.VectorSubcoreMesh(core_axis_name="c", subcore_axis_name="s")


def _build_pack(rows, cols):
    rpc = rows // NW
    wrows = rpc // 4

    def body(m_ref, mw_ref, pre_ref, run):
        g = pl.program_id(0)

        @pl.when(g == 0)
        def _():
            run[0] = 0

        mf = m_ref[...].astype(jnp.float32)
        dn = (((1,), (0,)), ((), ()))
        lo = jnp.zeros((wrows, 128), jnp.float32)
        hi = jnp.zeros((wrows, 128), jnp.float32)
        for s in range(4):
            ii = lax.broadcasted_iota(jnp.int32, (wrows, rpc), 0)
            jj = lax.broadcasted_iota(jnp.int32, (wrows, rpc), 1)
            a_s = (jj == 4 * ii + s).astype(jnp.float32)
            m_s = lax.dot_general(a_s, mf, dn, preferred_element_type=jnp.float32)
            cc = lax.broadcasted_iota(jnp.int32, (128, 128), 0)
            ll = lax.broadcasted_iota(jnp.int32, (128, 128), 1)
            hit = ll == 32 * s + lax.div(cc, 4)
            byte = cc % 4
            b_lo = jnp.where(hit & (byte == 0), 1.0, 0.0) + jnp.where(
                hit & (byte == 1), 256.0, 0.0)
            b_hi = jnp.where(hit & (byte == 2), 1.0, 0.0) + jnp.where(
                hit & (byte == 3), 256.0, 0.0)
            lo = lo + lax.dot_general(m_s, b_lo, dn, preferred_element_type=jnp.float32)
            hi = hi + lax.dot_general(m_s, b_hi, dn, preferred_element_type=jnp.float32)
        mw_ref[...] = lo.astype(jnp.int32) + lax.shift_left(hi.astype(jnp.int32), 16)
        pre_ref[...] = jnp.full((1, 1, 128), run[0], jnp.int32)
        run[0] += jnp.sum(mf).astype(jnp.int32)

    return pl.pallas_call(
        body,
        grid=(NW,),
        in_specs=[pl.BlockSpec((rpc, cols), lambda w: (w, 0))],
        out_specs=[
            pl.BlockSpec((wrows, 128), lambda w: (w, 0)),
            pl.BlockSpec((1, 1, 128), lambda w: (w, 0, 0)),
        ],
        out_shape=[
            jax.ShapeDtypeStruct((NW * wrows, 128), jnp.int32),
            jax.ShapeDtypeStruct((NW, 1, 128), jnp.int32),
        ],
        scratch_shapes=[pltpu.SMEM((1,), jnp.int32)],
    )


def _build_apply(n):
    chunk = n // NW
    sub = 32768
    subw = sub // 4
    nsub = chunk // sub
    srcv_len = sub + 16

    @functools.partial(
        pl.kernel,
        mesh=_mesh(),
        out_type=jax.ShapeDtypeStruct((n,), jnp.float32),
        compiler_params=pltpu.CompilerParams(needs_layout_passes=False),
        scratch_types=[
            pltpu.VMEM((subw,), jnp.int32),
            pltpu.VMEM((srcv_len,), jnp.float32),
            pltpu.VMEM((sub,), jnp.float32),
            pltpu.VMEM((L,), jnp.int32),
        ],
    )
    def k(mw_hbm, x_hbm, src_hbm, pre_hbm, out_hbm, mwv, srcv, outv, base_v):
        wid = lax.axis_index("s") * NC + lax.axis_index("c")
        poff = pl.multiple_of(wid * 128, 8)
        pltpu.sync_copy(pre_hbm.at[pl.ds(poff, L)], base_v)
        base = jnp.max(base_v[...])

        chunk_off = wid * chunk
        iota4 = lax.iota(jnp.int32, L) * 4

        def sub_body(b, pos):
            eoff = pl.multiple_of(chunk_off + b * sub, 8)
            pltpu.sync_copy(x_hbm.at[pl.ds(eoff, sub)], outv)
            woff = pl.multiple_of((chunk_off + b * sub) // 4, 8)
            pltpu.sync_copy(mw_hbm.at[pl.ds(woff, subw)], mwv)
            a = pl.multiple_of(jnp.minimum(pos & -8, n - srcv_len), 8)
            adj = pos - a
            pltpu.sync_copy(src_hbm.at[pl.ds(a, srcv_len)], srcv)

            def ibody(kk, c0):
                w = mwv[pl.ds(kk * L, L)]
                p = w * MULT
                t = lax.shift_right_logical(p, 24)
                excl = plsc.cumsum(t) - t
                eb = excl + c0
                i0 = p & 0xFF
                i1 = lax.shift_right_logical(p, 8) & 0xFF
                i2 = lax.shift_right_logical(p, 16) & 0xFF
                incs = (i0, i1, i2, t)
                ms = (i0 > 0, i1 > i0, i2 > i1, t > i2)
                xb = iota4 + kk * (4 * L)
                tot = None
                for j in range(4):
                    g = plsc.load_gather(srcv, [jnp.maximum(eb + incs[j], 0)])
                    plsc.store_scatter(outv, [xb + j], g, mask=ms[j])
                    pc = plsc.all_reduce_population_count(ms[j])
                    tot = pc if tot is None else tot + pc
                return c0 + tot

            c0f = lax.fori_loop(0, subw // L, ibody, jnp.full((L,), adj - 1, jnp.int32))
            pltpu.sync_copy(outv, out_hbm.at[pl.ds(eoff, sub)])
            return pos + (jnp.max(c0f) - (adj - 1))

        lax.fori_loop(0, nsub, sub_body, base)

    return k


def kernel(x, mask, source):
    n = x.size
    xf = x.reshape(-1)
    sf = source.reshape(-1)
    mw2d, pre = _build_pack(*mask.shape)(mask)
    out = _build_apply(n)(mw2d.reshape(-1), xf, sf, pre.reshape(-1))
    return out.reshape(x.shape)


# speedup vs baseline: 6.7518x; 1.1099x over previous
import functools

import jax
import jax.numpy as jnp
import numpy as np
from jax import lax
from jax.experimental import pallas as pl
from jax.experimental.pallas import tpu as pltpu
from jax.experimental.pallas import tpu_sc as plsc

L = 16
NC = 2
NS = 16
NW = NC * NS
MULT = 0x01010101


def _mesh():
    return plsc.VectorSubcoreMesh(core_axis_name="c", subcore_axis_name="s")


def _pack_consts():
    b_lo = np.zeros((4 * 128, 128), np.float32)
    b_hi = np.zeros((4 * 128, 128), np.float32)
    for s in range(4):
        for c in range(128):
            l = 32 * s + c // 4
            byte = c % 4
            if byte == 0:
                b_lo[128 * s + c, l] = 1.0
            elif byte == 1:
                b_lo[128 * s + c, l] = 256.0
            elif byte == 2:
                b_hi[128 * s + c, l] = 1.0
            else:
                b_hi[128 * s + c, l] = 256.0
    return jnp.asarray(b_lo, jnp.bfloat16), jnp.asarray(b_hi, jnp.bfloat16)


def _build_pack(rows, cols):
    rpc = rows // NW

    def body(m_ref, blo_ref, bhi_ref, mw_ref, pre_ref):
        dn = (((1,), (0,)), ((), ()))
        lo = jnp.zeros((rows // 4, 128), jnp.float32)
        hi = jnp.zeros((rows // 4, 128), jnp.float32)
        for s in range(4):
            m_s = m_ref[:, s, :].astype(jnp.bfloat16)
            lo = lo + lax.dot_general(
                m_s, blo_ref[pl.ds(128 * s, 128), :], dn,
                preferred_element_type=jnp.float32)
            hi = hi + lax.dot_general(
                m_s, bhi_ref[pl.ds(128 * s, 128), :], dn,
                preferred_element_type=jnp.float32)
        mw_ref[...] = lo.astype(jnp.int32) + lax.shift_left(hi.astype(jnp.int32), 16)
        wpc = rpc // 4
        run = 0
        for c in range(NW):
            pre_ref[c] = jnp.full((1, 128), run, jnp.int32)
            bs = lo[wpc * c:wpc * (c + 1), :] + hi[wpc * c:wpc * (c + 1), :]
            byt = bs.astype(jnp.int32)
            run = run + jnp.sum(
                (byt & 0xFF) + lax.shift_right_logical(byt, 8))

    return pl.pallas_call(
        body,
        out_shape=[
            jax.ShapeDtypeStruct((rows // 4, 128), jnp.int32),
            jax.ShapeDtypeStruct((NW, 1, 128), jnp.int32),
        ],
    )


def _build_apply(n):
    chunk = n // NW
    sub = 32768
    subw = sub // 4
    nsub = chunk // sub
    srcv_len = sub + 16

    @functools.partial(
        pl.kernel,
        mesh=_mesh(),
        out_type=jax.ShapeDtypeStruct((n,), jnp.float32),
        compiler_params=pltpu.CompilerParams(needs_layout_passes=False),
        scratch_types=[
            pltpu.VMEM((subw,), jnp.int32),
            pltpu.VMEM((srcv_len,), jnp.float32),
            pltpu.VMEM((sub,), jnp.float32),
            pltpu.VMEM((L,), jnp.int32),
        ],
    )
    def k(mw_hbm, x_hbm, src_hbm, pre_hbm, out_hbm, mwv, srcv, outv, base_v):
        wid = lax.axis_index("s") * NC + lax.axis_index("c")
        poff = pl.multiple_of(wid * 128, 8)
        pltpu.sync_copy(pre_hbm.at[pl.ds(poff, L)], base_v)
        base = jnp.max(base_v[...])

        chunk_off = wid * chunk
        iota4 = lax.iota(jnp.int32, L) * 4

        def sub_body(b, pos):
            eoff = pl.multiple_of(chunk_off + b * sub, 8)
            pltpu.sync_copy(x_hbm.at[pl.ds(eoff, sub)], outv)
            woff = pl.multiple_of((chunk_off + b * sub) // 4, 8)
            pltpu.sync_copy(mw_hbm.at[pl.ds(woff, subw)], mwv)
            a = pl.multiple_of(jnp.minimum(pos & -8, n - srcv_len), 8)
            adj = pos - a
            pltpu.sync_copy(src_hbm.at[pl.ds(a, srcv_len)], srcv)

            def ibody(kk, c0):
                w = mwv[pl.ds(kk * L, L)]
                p = w * MULT
                t = lax.shift_right_logical(p, 24)
                excl = plsc.cumsum(t) - t
                eb = excl + c0
                i0 = p & 0xFF
                i1 = lax.shift_right_logical(p, 8) & 0xFF
                i2 = lax.shift_right_logical(p, 16) & 0xFF
                incs = (i0, i1, i2, t)
                ms = (i0 > 0, i1 > i0, i2 > i1, t > i2)
                xb = iota4 + kk * (4 * L)
                tot = None
                for j in range(4):
                    g = plsc.load_gather(srcv, [jnp.maximum(eb + incs[j], 0)])
                    plsc.store_scatter(outv, [xb + j], g, mask=ms[j])
                    pc = plsc.all_reduce_population_count(ms[j])
                    tot = pc if tot is None else tot + pc
                return c0 + tot

            c0f = lax.fori_loop(0, subw // L, ibody, jnp.full((L,), adj - 1, jnp.int32))
            pltpu.sync_copy(outv, out_hbm.at[pl.ds(eoff, sub)])
            return pos + (jnp.max(c0f) - (adj - 1))

        lax.fori_loop(0, nsub, sub_body, base)

    return k


def kernel(x, mask, source):
    n = x.size
    xf = x.reshape(-1)
    sf = source.reshape(-1)
    blo, bhi = _pack_consts()
    m3 = mask.reshape(mask.shape[0] // 4, 4, mask.shape[1])
    mw2d, pre = _build_pack(*mask.shape)(m3, blo, bhi)
    out = _build_apply(n)(mw2d.reshape(-1), xf, sf, pre.reshape(-1))
    return out.reshape(x.shape)


# speedup vs baseline: 7.0919x; 1.0504x over previous
import functools

import jax
import jax.numpy as jnp
import numpy as np
from jax import lax
from jax.experimental import pallas as pl
from jax.experimental.pallas import tpu as pltpu
from jax.experimental.pallas import tpu_sc as plsc

L = 16
NC = 2
NS = 16
NW = NC * NS
MULT = 0x01010101


def _mesh():
    return plsc.VectorSubcoreMesh(core_axis_name="c", subcore_axis_name="s")


def _pack_consts():
    b_lo = np.zeros((4 * 128, 128), np.float32)
    b_hi = np.zeros((4 * 128, 128), np.float32)
    for s in range(4):
        for c in range(128):
            l = 32 * s + c // 4
            byte = c % 4
            if byte == 0:
                b_lo[128 * s + c, l] = 1.0
            elif byte == 1:
                b_lo[128 * s + c, l] = 256.0
            elif byte == 2:
                b_hi[128 * s + c, l] = 1.0
            else:
                b_hi[128 * s + c, l] = 256.0
    return jnp.asarray(b_lo, jnp.bfloat16), jnp.asarray(b_hi, jnp.bfloat16)


def _build_pack(rows, cols):
    rpc = rows // NW

    def body(m_ref, blo_ref, bhi_ref, mw_ref, pre_ref):
        dn = (((1,), (0,)), ((), ()))
        lo = jnp.zeros((rows // 4, 128), jnp.float32)
        hi = jnp.zeros((rows // 4, 128), jnp.float32)
        for s in range(4):
            m_s = m_ref[:, s, :].astype(jnp.bfloat16)
            lo = lo + lax.dot_general(
                m_s, blo_ref[pl.ds(128 * s, 128), :], dn,
                preferred_element_type=jnp.float32)
            hi = hi + lax.dot_general(
                m_s, bhi_ref[pl.ds(128 * s, 128), :], dn,
                preferred_element_type=jnp.float32)
        mw_ref[...] = lo.astype(jnp.int32) + lax.shift_left(hi.astype(jnp.int32), 16)
        wpc = rpc // 4
        run = 0
        for c in range(NW):
            pre_ref[c] = jnp.full((1, 128), run, jnp.int32)
            bs = lo[wpc * c:wpc * (c + 1), :] + hi[wpc * c:wpc * (c + 1), :]
            byt = bs.astype(jnp.int32)
            run = run + jnp.sum(
                (byt & 0xFF) + lax.shift_right_logical(byt, 8))

    return pl.pallas_call(
        body,
        out_shape=[
            jax.ShapeDtypeStruct((rows // 4, 128), jnp.int32),
            jax.ShapeDtypeStruct((NW, 1, 128), jnp.int32),
        ],
    )


def _build_apply(n):
    chunk = n // NW
    sub = 16384
    subw = sub // 4
    nsub = chunk // sub
    srcv_len = sub + 16

    @functools.partial(
        pl.kernel,
        mesh=_mesh(),
        out_type=jax.ShapeDtypeStruct((n,), jnp.float32),
        compiler_params=pltpu.CompilerParams(needs_layout_passes=False),
        scratch_types=[
            pltpu.VMEM((chunk // 4,), jnp.int32),
            pltpu.VMEM((srcv_len,), jnp.float32),
            pltpu.VMEM((srcv_len,), jnp.float32),
            pltpu.VMEM((sub,), jnp.float32),
            pltpu.VMEM((sub,), jnp.float32),
            pltpu.VMEM((L,), jnp.int32),
            pltpu.SemaphoreType.DMA,
            pltpu.SemaphoreType.DMA,
            pltpu.SemaphoreType.DMA,
            pltpu.SemaphoreType.DMA,
            pltpu.SemaphoreType.DMA,
            pltpu.SemaphoreType.DMA,
        ],
    )
    def k(mw_hbm, x_hbm, src_hbm, pre_hbm, out_hbm, mwv, srcv0, srcv1,
          outv0, outv1, base_v, sx0, sx1, ss0, ss1, so0, so1):
        srcv = (srcv0, srcv1)
        outv = (outv0, outv1)
        semx = (sx0, sx1)
        sems = (ss0, ss1)
        semo = (so0, so1)
        wid = lax.axis_index("s") * NC + lax.axis_index("c")
        poff = pl.multiple_of(wid * 128, 8)
        pltpu.sync_copy(pre_hbm.at[pl.ds(poff, L)], base_v)
        base = jnp.max(base_v[...])
        chunk_off = wid * chunk
        iota4 = lax.iota(jnp.int32, L) * 4

        def x_start(b, slot):
            eoff = pl.multiple_of(chunk_off + b * sub, 8)
            return pltpu.async_copy(
                x_hbm.at[pl.ds(eoff, sub)], outv[slot], semx[slot])

        def src_start(pos, slot):
            a = pl.multiple_of(jnp.minimum(pos & -8, n - srcv_len), 8)
            return pltpu.async_copy(
                src_hbm.at[pl.ds(a, srcv_len)], srcv[slot], sems[slot]), pos - a

        hx = [None] * nsub
        hs = [None] * nsub
        ho = [None] * nsub
        adjs = [None] * nsub
        hx[0] = x_start(0, 0)
        hs[0], adjs[0] = src_start(base, 0)
        woff = pl.multiple_of(chunk_off // 4, 8)
        pltpu.sync_copy(mw_hbm.at[pl.ds(woff, chunk // 4)], mwv)

        offs = [base]
        for b in range(nsub - 1):
            def cbody(i, acc):
                w = mwv[pl.ds(b * subw + i * L, L)]
                return acc + lax.shift_right_logical(w * MULT, 24)

            acc = lax.fori_loop(0, subw // L, cbody, jnp.zeros((L,), jnp.int32))
            offs.append(offs[b] + jnp.sum(acc))

        for b in range(nsub):
            if b + 1 < nsub:
                if b >= 1:
                    ho[b - 1].wait()
                hx[b + 1] = x_start(b + 1, (b + 1) % 2)
                hs[b + 1], adjs[b + 1] = src_start(offs[b + 1], (b + 1) % 2)
            hx[b].wait()
            hs[b].wait()
            sv = srcv[b % 2]
            ov = outv[b % 2]

            def ibody(kk, c0):
                w = mwv[pl.ds(b * subw + kk * L, L)]
                p = w * MULT
                t = lax.shift_right_logical(p, 24)
                excl = plsc.cumsum(t) - t
                eb = excl + c0
                i0 = p & 0xFF
                i1 = lax.shift_right_logical(p, 8) & 0xFF
                i2 = lax.shift_right_logical(p, 16) & 0xFF
                incs = (i0, i1, i2, t)
                ms = (i0 > 0, i1 > i0, i2 > i1, t > i2)
                xb = iota4 + kk * (4 * L)
                tot = None
                for j in range(4):
                    g = plsc.load_gather(sv, [jnp.maximum(eb + incs[j], 0)])
                    plsc.store_scatter(ov, [xb + j], g, mask=ms[j])
                    pc = plsc.all_reduce_population_count(ms[j])
                    tot = pc if tot is None else tot + pc
                return c0 + tot

            lax.fori_loop(0, subw // L, ibody,
                          jnp.full((L,), adjs[b] - 1, jnp.int32))
            eoff = pl.multiple_of(chunk_off + b * sub, 8)
            ho[b] = pltpu.async_copy(ov, out_hbm.at[pl.ds(eoff, sub)],
                                     semo[b % 2])
        ho[nsub - 2].wait()
        ho[nsub - 1].wait()

    return k


def kernel(x, mask, source):
    n = x.size
    xf = x.reshape(-1)
    sf = source.reshape(-1)
    blo, bhi = _pack_consts()
    m3 = mask.reshape(mask.shape[0] // 4, 4, mask.shape[1])
    mw2d, pre = _build_pack(*mask.shape)(m3, blo, bhi)
    out = _build_apply(n)(mw2d.reshape(-1), xf, sf, pre.reshape(-1))
    return out.reshape(x.shape)


# speedup vs baseline: 8.3838x; 1.1822x over previous
import functools

import jax
import jax.numpy as jnp
import numpy as np
from jax import lax
from jax.experimental import pallas as pl
from jax.experimental.pallas import tpu as pltpu
from jax.experimental.pallas import tpu_sc as plsc

L = 16
NC = 2
NS = 16
NW = NC * NS
MULT = 0x01010101


def _mesh():
    return plsc.VectorSubcoreMesh(core_axis_name="c", subcore_axis_name="s")


def _pack_consts():
    b_lo = np.zeros((4 * 128, 128), np.float32)
    b_hi = np.zeros((4 * 128, 128), np.float32)
    for s in range(4):
        for c in range(128):
            l = 32 * s + c // 4
            byte = c % 4
            if byte == 0:
                b_lo[128 * s + c, l] = 1.0
            elif byte == 1:
                b_lo[128 * s + c, l] = 256.0
            elif byte == 2:
                b_hi[128 * s + c, l] = 1.0
            else:
                b_hi[128 * s + c, l] = 256.0
    return jnp.asarray(b_lo, jnp.bfloat16), jnp.asarray(b_hi, jnp.bfloat16)


def _build_pack(rows, cols):
    rpc = rows // NW

    def body(m_ref, blo_ref, bhi_ref, mw_ref, pre_ref):
        dn = (((1,), (0,)), ((), ()))
        lo = jnp.zeros((rows // 4, 128), jnp.float32)
        hi = jnp.zeros((rows // 4, 128), jnp.float32)
        for s in range(4):
            m_s = m_ref[:, s, :].astype(jnp.bfloat16)
            lo = lo + lax.dot_general(
                m_s, blo_ref[pl.ds(128 * s, 128), :], dn,
                preferred_element_type=jnp.float32)
            hi = hi + lax.dot_general(
                m_s, bhi_ref[pl.ds(128 * s, 128), :], dn,
                preferred_element_type=jnp.float32)
        mw_ref[...] = lo.astype(jnp.int32) + lax.shift_left(hi.astype(jnp.int32), 16)
        wpc = rpc // 4
        run = 0
        for c in range(NW):
            pre_ref[c] = jnp.full((1, 128), run, jnp.int32)
            bs = lo[wpc * c:wpc * (c + 1), :] + hi[wpc * c:wpc * (c + 1), :]
            byt = bs.astype(jnp.int32)
            run = run + jnp.sum(
                (byt & 0xFF) + lax.shift_right_logical(byt, 8))

    return pl.pallas_call(
        body,
        out_shape=[
            jax.ShapeDtypeStruct((rows // 4, 128), jnp.int32),
            jax.ShapeDtypeStruct((NW, 1, 128), jnp.int32),
        ],
    )


def _build_apply(n):
    chunk = n // NW
    sub = 16384
    subw = sub // 4
    nsub = chunk // sub
    srcv_len = sub + 16

    @functools.partial(
        pl.kernel,
        mesh=_mesh(),
        out_type=jax.ShapeDtypeStruct((n,), jnp.float32),
        compiler_params=pltpu.CompilerParams(needs_layout_passes=False),
        scratch_types=[
            pltpu.VMEM((chunk // 4,), jnp.int32),
            pltpu.VMEM((srcv_len,), jnp.float32),
            pltpu.VMEM((srcv_len,), jnp.float32),
            pltpu.VMEM((sub,), jnp.float32),
            pltpu.VMEM((sub,), jnp.float32),
            pltpu.VMEM((L,), jnp.int32),
            pltpu.SemaphoreType.DMA,
            pltpu.SemaphoreType.DMA,
            pltpu.SemaphoreType.DMA,
            pltpu.SemaphoreType.DMA,
            pltpu.SemaphoreType.DMA,
            pltpu.SemaphoreType.DMA,
        ],
    )
    def k(mw_hbm, x_hbm, src_hbm, pre_hbm, out_hbm, mwv, srcv0, srcv1,
          outv0, outv1, base_v, sx0, sx1, ss0, ss1, so0, so1):
        srcv = (srcv0, srcv1)
        outv = (outv0, outv1)
        semx = (sx0, sx1)
        sems = (ss0, ss1)
        semo = (so0, so1)
        wid = lax.axis_index("s") * NC + lax.axis_index("c")
        poff = pl.multiple_of(wid * 128, 8)
        pltpu.sync_copy(pre_hbm.at[pl.ds(poff, L)], base_v)
        base = jnp.max(base_v[...])
        chunk_off = wid * chunk
        iota4 = lax.iota(jnp.int32, L) * 4

        def x_start(b, slot):
            eoff = pl.multiple_of(chunk_off + b * sub, 8)
            return pltpu.async_copy(
                x_hbm.at[pl.ds(eoff, sub)], outv[slot], semx[slot])

        def src_start(pos, slot):
            a = pl.multiple_of(jnp.minimum(pos & -8, n - srcv_len), 8)
            return pltpu.async_copy(
                src_hbm.at[pl.ds(a, srcv_len)], srcv[slot], sems[slot]), pos - a

        hx = [None] * nsub
        hs = [None] * nsub
        ho = [None] * nsub
        adjs = [None] * nsub
        hx[0] = x_start(0, 0)
        hs[0], adjs[0] = src_start(base, 0)
        woff = pl.multiple_of(chunk_off // 4, 8)
        pltpu.sync_copy(mw_hbm.at[pl.ds(woff, chunk // 4)], mwv)

        offs = [base]
        for b in range(nsub - 1):
            def cbody(i, accs):
                new = []
                for u in range(4):
                    w = mwv[pl.ds(b * subw + i * 4 * L + u * L, L)]
                    new.append(accs[u] + lax.shift_right_logical(w * MULT, 24))
                return tuple(new)

            accs = lax.fori_loop(0, subw // L // 4, cbody,
                                 tuple(jnp.zeros((L,), jnp.int32) for _ in range(4)))
            offs.append(offs[b] + jnp.sum(accs[0] + accs[1] + accs[2] + accs[3]))

        for b in range(nsub):
            if b + 1 < nsub:
                if b >= 1:
                    ho[b - 1].wait()
                hx[b + 1] = x_start(b + 1, (b + 1) % 2)
                hs[b + 1], adjs[b + 1] = src_start(offs[b + 1], (b + 1) % 2)
            hx[b].wait()
            hs[b].wait()
            sv = srcv[b % 2]
            ov = outv[b % 2]

            def ibody(kk, c0):
                pre = []
                for u in range(4):
                    w = mwv[pl.ds(b * subw + kk * 4 * L + u * L, L)]
                    p = w * MULT
                    t = lax.shift_right_logical(p, 24)
                    excl = plsc.cumsum(t) - t
                    i0 = p & 0xFF
                    i1 = lax.shift_right_logical(p, 8) & 0xFF
                    i2 = lax.shift_right_logical(p, 16) & 0xFF
                    incs = (i0, i1, i2, t)
                    ms = (i0 > 0, i1 > i0, i2 > i1, t > i2)
                    tot = None
                    for j in range(4):
                        pc = plsc.all_reduce_population_count(ms[j])
                        tot = pc if tot is None else tot + pc
                    pre.append((excl, incs, ms, tot))
                xb0 = iota4 + kk * (16 * L)
                c = c0
                for u in range(4):
                    excl, incs, ms, tot = pre[u]
                    eb = excl + c
                    xb = xb0 + u * (4 * L)
                    for j in range(4):
                        g = plsc.load_gather(sv, [jnp.maximum(eb + incs[j], 0)])
                        plsc.store_scatter(ov, [xb + j], g, mask=ms[j])
                    c = c + tot
                return c

            lax.fori_loop(0, subw // L // 4, ibody,
                          jnp.full((L,), adjs[b] - 1, jnp.int32))
            eoff = pl.multiple_of(chunk_off + b * sub, 8)
            ho[b] = pltpu.async_copy(ov, out_hbm.at[pl.ds(eoff, sub)],
                                     semo[b % 2])
        ho[nsub - 2].wait()
        ho[nsub - 1].wait()

    return k


def kernel(x, mask, source):
    n = x.size
    xf = x.reshape(-1)
    sf = source.reshape(-1)
    blo, bhi = _pack_consts()
    m3 = mask.reshape(mask.shape[0] // 4, 4, mask.shape[1])
    mw2d, pre = _build_pack(*mask.shape)(m3, blo, bhi)
    out = _build_apply(n)(mw2d.reshape(-1), xf, sf, pre.reshape(-1))
    return out.reshape(x.shape)


# speedup vs baseline: 8.6598x; 1.0329x over previous
import functools

import jax
import jax.numpy as jnp
import numpy as np
from jax import lax
from jax.experimental import pallas as pl
from jax.experimental.pallas import tpu as pltpu
from jax.experimental.pallas import tpu_sc as plsc

L = 16
NC = 2
NS = 16
NW = NC * NS
MULT = 0x01010101


def _mesh():
    return plsc.VectorSubcoreMesh(core_axis_name="c", subcore_axis_name="s")


def _pack_consts():
    b_lo = np.zeros((4 * 128, 128), np.float32)
    b_hi = np.zeros((4 * 128, 128), np.float32)
    for s in range(4):
        for c in range(128):
            l = 32 * s + c // 4
            byte = c % 4
            if byte == 0:
                b_lo[128 * s + c, l] = 1.0
            elif byte == 1:
                b_lo[128 * s + c, l] = 256.0
            elif byte == 2:
                b_hi[128 * s + c, l] = 1.0
            else:
                b_hi[128 * s + c, l] = 256.0
    return jnp.asarray(b_lo, jnp.bfloat16), jnp.asarray(b_hi, jnp.bfloat16)


def _build_pack(rows, cols):
    rpc = rows // NW

    def body(m_ref, blo_ref, bhi_ref, mw_ref, pre_ref):
        dn = (((1,), (0,)), ((), ()))
        lo = jnp.zeros((rows // 4, 128), jnp.float32)
        hi = jnp.zeros((rows // 4, 128), jnp.float32)
        for s in range(4):
            m_s = m_ref[:, s, :].astype(jnp.bfloat16)
            lo = lo + lax.dot_general(
                m_s, blo_ref[pl.ds(128 * s, 128), :], dn,
                preferred_element_type=jnp.float32)
            hi = hi + lax.dot_general(
                m_s, bhi_ref[pl.ds(128 * s, 128), :], dn,
                preferred_element_type=jnp.float32)
        mw_ref[...] = lo.astype(jnp.int32) + lax.shift_left(hi.astype(jnp.int32), 16)
        wpc = rpc // 4
        run = 0
        for c in range(NW):
            pre_ref[c] = jnp.full((1, 128), run, jnp.int32)
            bs = lo[wpc * c:wpc * (c + 1), :] + hi[wpc * c:wpc * (c + 1), :]
            byt = bs.astype(jnp.int32)
            run = run + jnp.sum(
                (byt & 0xFF) + lax.shift_right_logical(byt, 8))

    return pl.pallas_call(
        body,
        out_shape=[
            jax.ShapeDtypeStruct((rows // 4, 128), jnp.int32),
            jax.ShapeDtypeStruct((NW, 1, 128), jnp.int32),
        ],
    )


def _build_apply(n):
    chunk = n // NW
    sub = 16384
    subw = sub // 4
    nsub = chunk // sub
    srcv_len = sub + 16

    @functools.partial(
        pl.kernel,
        mesh=_mesh(),
        out_type=jax.ShapeDtypeStruct((n,), jnp.float32),
        compiler_params=pltpu.CompilerParams(needs_layout_passes=False),
        scratch_types=[
            pltpu.VMEM((chunk // 4,), jnp.int32),
            pltpu.VMEM((srcv_len,), jnp.float32),
            pltpu.VMEM((srcv_len,), jnp.float32),
            pltpu.VMEM((sub,), jnp.float32),
            pltpu.VMEM((sub,), jnp.float32),
            pltpu.VMEM((L,), jnp.int32),
            pltpu.SemaphoreType.DMA,
            pltpu.SemaphoreType.DMA,
            pltpu.SemaphoreType.DMA,
            pltpu.SemaphoreType.DMA,
            pltpu.SemaphoreType.DMA,
            pltpu.SemaphoreType.DMA,
        ],
    )
    def k(mw_hbm, x_hbm, src_hbm, pre_hbm, out_hbm, mwv, srcv0, srcv1,
          outv0, outv1, base_v, sx0, sx1, ss0, ss1, so0, so1):
        srcv = (srcv0, srcv1)
        outv = (outv0, outv1)
        semx = (sx0, sx1)
        sems = (ss0, ss1)
        semo = (so0, so1)
        wid = lax.axis_index("s") * NC + lax.axis_index("c")
        poff = pl.multiple_of(wid * 128, 8)
        pltpu.sync_copy(pre_hbm.at[pl.ds(poff, L)], base_v)
        base = jnp.max(base_v[...])
        chunk_off = wid * chunk
        iota4 = lax.iota(jnp.int32, L) * 4

        def x_start(b, slot):
            eoff = pl.multiple_of(chunk_off + b * sub, 8)
            return pltpu.async_copy(
                x_hbm.at[pl.ds(eoff, sub)], outv[slot], semx[slot])

        def src_start(pos, slot):
            a = pl.multiple_of(jnp.minimum(pos & -8, n - srcv_len), 8)
            return pltpu.async_copy(
                src_hbm.at[pl.ds(a, srcv_len)], srcv[slot], sems[slot]), pos - a

        hx = [None] * nsub
        hs = [None] * nsub
        ho = [None] * nsub
        adjs = [None] * nsub
        hx[0] = x_start(0, 0)
        hs[0], adjs[0] = src_start(base, 0)
        woff = pl.multiple_of(chunk_off // 4, 8)
        pltpu.sync_copy(mw_hbm.at[pl.ds(woff, chunk // 4)], mwv)

        offs = [base]
        for b in range(nsub - 1):
            def cbody(i, accs):
                new = []
                for u in range(4):
                    w = mwv[pl.ds(b * subw + i * 4 * L + u * L, L)]
                    new.append(accs[u] + lax.shift_right_logical(w * MULT, 24))
                return tuple(new)

            accs = lax.fori_loop(0, subw // L // 4, cbody,
                                 tuple(jnp.zeros((L,), jnp.int32) for _ in range(4)))
            offs.append(offs[b] + jnp.sum(accs[0] + accs[1] + accs[2] + accs[3]))

        for b in range(nsub):
            if b + 1 < nsub:
                if b >= 1:
                    ho[b - 1].wait()
                hx[b + 1] = x_start(b + 1, (b + 1) % 2)
                hs[b + 1], adjs[b + 1] = src_start(offs[b + 1], (b + 1) % 2)
            hx[b].wait()
            hs[b].wait()
            sv = srcv[b % 2]
            ov = outv[b % 2]

            def ibody(kk, c0):
                pre = []
                for u in range(8):
                    w = mwv[pl.ds(b * subw + kk * 8 * L + u * L, L)]
                    p = w * MULT
                    t = lax.shift_right_logical(p, 24)
                    excl = plsc.cumsum(t) - t
                    i0 = p & 0xFF
                    i1 = lax.shift_right_logical(p, 8) & 0xFF
                    i2 = lax.shift_right_logical(p, 16) & 0xFF
                    incs = (i0, i1, i2, t)
                    ms = (i0 > 0, i1 > i0, i2 > i1, t > i2)
                    tot = None
                    for j in range(4):
                        pc = plsc.all_reduce_population_count(ms[j])
                        tot = pc if tot is None else tot + pc
                    pre.append((excl, incs, ms, tot))
                xb0 = iota4 + kk * (32 * L)
                c = c0
                for u in range(8):
                    excl, incs, ms, tot = pre[u]
                    eb = excl + c
                    xb = xb0 + u * (4 * L)
                    for j in range(4):
                        g = plsc.load_gather(sv, [jnp.maximum(eb + incs[j], 0)])
                        plsc.store_scatter(ov, [xb + j], g, mask=ms[j])
                    c = c + tot
                return c

            lax.fori_loop(0, subw // L // 8, ibody,
                          jnp.full((L,), adjs[b] - 1, jnp.int32))
            eoff = pl.multiple_of(chunk_off + b * sub, 8)
            ho[b] = pltpu.async_copy(ov, out_hbm.at[pl.ds(eoff, sub)],
                                     semo[b % 2])
        ho[nsub - 2].wait()
        ho[nsub - 1].wait()

    return k


def kernel(x, mask, source):
    n = x.size
    xf = x.reshape(-1)
    sf = source.reshape(-1)
    blo, bhi = _pack_consts()
    m3 = mask.reshape(mask.shape[0] // 4, 4, mask.shape[1])
    mw2d, pre = _build_pack(*mask.shape)(m3, blo, bhi)
    out = _build_apply(n)(mw2d.reshape(-1), xf, sf, pre.reshape(-1))
    return out.reshape(x.shape)
